# Initial kernel scaffold; baseline (speedup 1.0000x reference)
import jax
import jax.numpy as jnp
from jax.experimental import pallas as pl


def kernel(X, T, hyperedge_index, cW1, cb1, cW2, cb2, lin_W, att, conv_bias, f0W1, f0b1, f0W2, f0b2, f1W1, f1b1, f1W2, f1b2):
    raise NotImplementedError("write your pallas kernel here")


# speedup vs baseline: 33.4575x; 33.4575x over previous
import functools

import jax
import jax.numpy as jnp
from jax import lax
from jax.experimental import pallas as pl
from jax.experimental.pallas import tpu as pltpu
from jax.experimental.pallas import tpu_sc as plsc

N = 10000
E = 10000
I = 320000
NIN = 128
HD = 64
GD = 64
HEADS = 2
XLW = HEADS * GD

NC, NS = 2, 16
NW = NC * NS
PER_TILE = I // NW
C = 80
NCH = PER_TILE // C
RB = 624
ZR = 208
TAIL = N - RB * NS

F32 = jnp.float32
_mesh = plsc.VectorSubcoreMesh(core_axis_name="c", subcore_axis_name="s",
                               num_cores=NC, num_subcores=NS)
_sc_params = pltpu.CompilerParams(use_tc_tiling_on_sc=False)

def _fill(ref, ncol16, val):
    val_vec = jnp.full((16,), val, F32)

    def body(i, carry):
        for t in range(ncol16):
            ref[i, pl.ds(t * 16, 16)] = val_vec
        return carry
    lax.fori_loop(0, ref.shape[0], body, 0)


def _worker_id():
    return lax.axis_index("s") * NC + lax.axis_index("c")


def _zero_shared(zb, sh, sid):
    for t in range(3):
        pltpu.sync_copy(zb, sh.at[pl.ds(sid * RB + t * ZR, ZR)])

    @pl.when(sid == 0)
    def _():
        pltpu.sync_copy(zb.at[pl.ds(0, TAIL)], sh.at[pl.ds(RB * NS, TAIL)])


def _writeback(sh, out, cid, sid):
    for t in range(3):
        o = sid * RB + t * ZR
        pltpu.sync_copy(sh.at[pl.ds(o, ZR)], out.at[cid, pl.ds(o, ZR)])

    @pl.when(sid == 0)
    def _():
        pltpu.sync_copy(sh.at[pl.ds(RB * NS, TAIL)],
                        out.at[cid, pl.ds(RB * NS, TAIL)])


def _k2_body(p_hbm, ni_hbm, he_hbm,
             hes_out, cnt_out, deg_out,
             he_sh, cnt_sh, deg_sh,
             zb64, zb16, ones16, ni_v, he_v, rows_v, sem):
    cid = lax.axis_index("c")
    sid = lax.axis_index("s")
    w = _worker_id()
    _fill(zb64, 4, 0.0)
    _fill(zb16, 1, 0.0)
    _fill(ones16, 1, 1.0)
    _zero_shared(zb64, he_sh, sid)
    _zero_shared(zb16, cnt_sh, sid)
    _zero_shared(zb16, deg_sh, sid)
    plsc.subcore_barrier()
    base = w * PER_TILE

    def chunk(g, carry):
        off = base + g * C
        pltpu.sync_copy(ni_hbm.at[pl.ds(off, C)], ni_v)
        pltpu.sync_copy(he_hbm.at[pl.ds(off, C)], he_v)
        pltpu.async_copy(p_hbm.at[ni_v], rows_v, sem).wait()
        pltpu.sync_copy(rows_v, he_sh.at[he_v], add=True)
        pltpu.sync_copy(ones16, cnt_sh.at[he_v], add=True)
        pltpu.sync_copy(ones16, deg_sh.at[ni_v], add=True)
        return carry

    lax.fori_loop(0, NCH, chunk, 0)
    plsc.subcore_barrier()
    _writeback(he_sh, hes_out, cid, sid)
    _writeback(cnt_sh, cnt_out, cid, sid)
    _writeback(deg_sh, deg_out, cid, sid)


_k2 = pl.kernel(
    _k2_body,
    compiler_params=_sc_params,
    out_type=(
        jax.ShapeDtypeStruct((NC, E, HD), F32),
        jax.ShapeDtypeStruct((NC, E, 16), F32),
        jax.ShapeDtypeStruct((NC, N, 16), F32),
    ),
    mesh=_mesh,
    scratch_types=[
        pltpu.VMEM_SHARED((E, HD), F32),
        pltpu.VMEM_SHARED((E, 16), F32),
        pltpu.VMEM_SHARED((N, 16), F32),
        pltpu.VMEM((ZR, HD), F32),
        pltpu.VMEM((ZR, 16), F32),
        pltpu.VMEM((C, 16), F32),
        pltpu.VMEM((C,), jnp.int32),
        pltpu.VMEM((C,), jnp.int32),
        pltpu.VMEM((C, HD), F32),
        pltpu.SemaphoreType.DMA,
    ],
)


def _k4_body(a_hbm, b_hbm, amax_hbm, ni_hbm, he_hbm,
             ex_out, esum_out,
             es_sh, zb16, amax_v, ni_v, he_v, ag, bg, exb, sem):
    cid = lax.axis_index("c")
    sid = lax.axis_index("s")
    w = _worker_id()
    _fill(zb16, 1, 0.0)
    pltpu.sync_copy(amax_hbm, amax_v)
    _zero_shared(zb16, es_sh, sid)
    plsc.subcore_barrier()
    amx = amax_v[...]
    base = w * PER_TILE

    def chunk(g, carry):
        off = base + g * C
        pltpu.sync_copy(ni_hbm.at[pl.ds(off, C)], ni_v)
        pltpu.sync_copy(he_hbm.at[pl.ds(off, C)], he_v)
        d1 = pltpu.async_copy(a_hbm.at[ni_v], ag, sem)
        d2 = pltpu.async_copy(b_hbm.at[he_v], bg, sem)
        d1.wait()
        d2.wait()

        def row(j, c2):
            av = ag[j]
            bv = bg[j]
            s = av + bv
            al = jnp.maximum(s, 0.2 * s)
            m = bv + amx
            ml = jnp.maximum(m, 0.2 * m)
            exb[j] = jnp.exp(al - ml)
            return c2

        lax.fori_loop(0, C, row, 0)
        pltpu.sync_copy(exb, ex_out.at[pl.ds(off, C)])
        pltpu.sync_copy(exb, es_sh.at[he_v], add=True)
        return carry

    lax.fori_loop(0, NCH, chunk, 0)
    plsc.subcore_barrier()
    _writeback(es_sh, esum_out, cid, sid)


_k4 = pl.kernel(
    _k4_body,
    compiler_params=_sc_params,
    out_type=(
        jax.ShapeDtypeStruct((I, 16), F32),
        jax.ShapeDtypeStruct((NC, E, 16), F32),
    ),
    mesh=_mesh,
    scratch_types=[
        pltpu.VMEM_SHARED((E, 16), F32),
        pltpu.VMEM((ZR, 16), F32),
        pltpu.VMEM((16,), F32),
        pltpu.VMEM((C,), jnp.int32),
        pltpu.VMEM((C,), jnp.int32),
        pltpu.VMEM((C, 16), F32),
        pltpu.VMEM((C, 16), F32),
        pltpu.VMEM((C, 16), F32),
        pltpu.SemaphoreType.DMA,
    ],
)


def _k5_body(xl_hbm, ex_hbm, es0_hbm, es1_hbm, ni_hbm, he_hbm,
             alpha_out, oe_out,
             oe_sh, zb, ni_v, he_v, xg, s0, s1, exv, alb, sem):
    cid = lax.axis_index("c")
    sid = lax.axis_index("s")
    w = _worker_id()
    _fill(zb, 8, 0.0)
    _zero_shared(zb, oe_sh, sid)
    plsc.subcore_barrier()
    base = w * PER_TILE

    def chunk(g, carry):
        off = base + g * C
        pltpu.sync_copy(ni_hbm.at[pl.ds(off, C)], ni_v)
        pltpu.sync_copy(he_hbm.at[pl.ds(off, C)], he_v)
        d1 = pltpu.async_copy(xl_hbm.at[ni_v], xg, sem)
        d2 = pltpu.async_copy(es0_hbm.at[he_v], s0, sem)
        d3 = pltpu.async_copy(es1_hbm.at[he_v], s1, sem)
        pltpu.sync_copy(ex_hbm.at[pl.ds(off, C)], exv)
        d1.wait()
        d2.wait()
        d3.wait()

        def row(j, c2):
            al = exv[j] / (s0[j] + s1[j] + 1e-16)
            alb[j] = al
            b0 = jnp.full((16,), al[0], F32)
            b1 = jnp.full((16,), al[1], F32)
            for t in range(8):
                bv = b0 if t < 4 else b1
                xg[j, pl.ds(t * 16, 16)] = xg[j, pl.ds(t * 16, 16)] * bv
            return c2

        lax.fori_loop(0, C, row, 0)
        pltpu.sync_copy(alb, alpha_out.at[pl.ds(off, C)])
        pltpu.sync_copy(xg, oe_sh.at[he_v], add=True)
        return carry

    lax.fori_loop(0, NCH, chunk, 0)
    plsc.subcore_barrier()
    _writeback(oe_sh, oe_out, cid, sid)


_k5 = pl.kernel(
    _k5_body,
    compiler_params=_sc_params,
    out_type=(
        jax.ShapeDtypeStruct((I, 16), F32),
        jax.ShapeDtypeStruct((NC, E, XLW), F32),
    ),
    mesh=_mesh,
    scratch_types=[
        pltpu.VMEM_SHARED((E, XLW), F32),
        pltpu.VMEM((ZR, XLW), F32),
        pltpu.VMEM((C,), jnp.int32),
        pltpu.VMEM((C,), jnp.int32),
        pltpu.VMEM((C, XLW), F32),
        pltpu.VMEM((C, 16), F32),
        pltpu.VMEM((C, 16), F32),
        pltpu.VMEM((C, 16), F32),
        pltpu.VMEM((C, 16), F32),
        pltpu.SemaphoreType.DMA,
    ],
)


def _k6_body(oe_hbm, al_hbm, ni_hbm, he_hbm,
             out_out,
             o_sh, zb, ni_v, he_v, xg, alb, sem):
    cid = lax.axis_index("c")
    sid = lax.axis_index("s")
    w = _worker_id()
    _fill(zb, 8, 0.0)
    _zero_shared(zb, o_sh, sid)
    plsc.subcore_barrier()
    base = w * PER_TILE

    def chunk(g, carry):
        off = base + g * C
        pltpu.sync_copy(ni_hbm.at[pl.ds(off, C)], ni_v)
        pltpu.sync_copy(he_hbm.at[pl.ds(off, C)], he_v)
        d1 = pltpu.async_copy(oe_hbm.at[he_v], xg, sem)
        pltpu.sync_copy(al_hbm.at[pl.ds(off, C)], alb)
        d1.wait()

        def row(j, c2):
            al = alb[j]
            b0 = jnp.full((16,), al[0], F32)
            b1 = jnp.full((16,), al[1], F32)
            for t in range(8):
                bv = b0 if t < 4 else b1
                xg[j, pl.ds(t * 16, 16)] = xg[j, pl.ds(t * 16, 16)] * bv
            return c2

        lax.fori_loop(0, C, row, 0)
        pltpu.sync_copy(xg, o_sh.at[ni_v], add=True)
        return carry

    lax.fori_loop(0, NCH, chunk, 0)
    plsc.subcore_barrier()
    _writeback(o_sh, out_out, cid, sid)


_k6 = pl.kernel(
    _k6_body,
    compiler_params=_sc_params,
    out_type=jax.ShapeDtypeStruct((NC, N, XLW), F32),
    mesh=_mesh,
    scratch_types=[
        pltpu.VMEM_SHARED((N, XLW), F32),
        pltpu.VMEM((ZR, XLW), F32),
        pltpu.VMEM((C,), jnp.int32),
        pltpu.VMEM((C,), jnp.int32),
        pltpu.VMEM((C, XLW), F32),
        pltpu.VMEM((C, 16), F32),
        pltpu.SemaphoreType.DMA,
    ],
)


BN = 2000
GRID = N // BN


def _k1_body(x_ref, tf_ref, cw1_ref, cb1_ref, cw2_ref, cb2_ref, linw_ref,
             attn_ref, z_ref, p_ref, xl_ref, an_ref, amax_ref):
    x = x_ref[...]
    z = jnp.maximum(jnp.dot(x, cw1_ref[...], preferred_element_type=F32)
                    + cb1_ref[...], 0.0)
    z = jnp.dot(z, cw2_ref[...], preferred_element_type=F32) + cb2_ref[...]
    z_ref[...] = z
    p = z * tf_ref[...]
    p_ref[...] = p
    xl = jnp.dot(p, linw_ref[...], preferred_element_type=F32)
    xl_ref[...] = xl
    an = jnp.dot(xl, attn_ref[...], preferred_element_type=F32)
    an_ref[...] = an
    cur = jnp.broadcast_to(jnp.max(an, axis=0, keepdims=True), (8, 16))

    @pl.when(pl.program_id(0) == 0)
    def _init():
        amax_ref[...] = cur

    @pl.when(pl.program_id(0) > 0)
    def _acc():
        amax_ref[...] = jnp.maximum(amax_ref[...], cur)


def _const_spec(shape):
    return pl.BlockSpec(shape, lambda i: (0, 0))


_k1 = pl.pallas_call(
    _k1_body,
    grid=(GRID,),
    in_specs=[
        pl.BlockSpec((BN, NIN), lambda i: (i, 0)),
        pl.BlockSpec((BN, 1), lambda i: (i, 0)),
        _const_spec((NIN, HD)),
        _const_spec((1, HD)),
        _const_spec((HD, HD)),
        _const_spec((1, HD)),
        _const_spec((HD, XLW)),
        _const_spec((XLW, 16)),
    ],
    out_specs=[
        pl.BlockSpec((BN, HD), lambda i: (i, 0)),
        pl.BlockSpec((BN, HD), lambda i: (i, 0)),
        pl.BlockSpec((BN, XLW), lambda i: (i, 0)),
        pl.BlockSpec((BN, 16), lambda i: (i, 0)),
        _const_spec((8, 16)),
    ],
    out_shape=[
        jax.ShapeDtypeStruct((N, HD), F32),
        jax.ShapeDtypeStruct((N, HD), F32),
        jax.ShapeDtypeStruct((N, XLW), F32),
        jax.ShapeDtypeStruct((N, 16), F32),
        jax.ShapeDtypeStruct((8, 16), F32),
    ],
)


def _k3_body(hs0_ref, hs1_ref, c0_ref, c1_ref, d0_ref, d1_ref, linw_ref,
             atte_ref, b_ref, binv_ref, dinv_ref):
    c = c0_ref[...][:, :1] + c1_ref[...][:, :1]
    crec = jnp.where(c > 0, 1.0 / jnp.where(c > 0, c, 1.0), 0.0)
    he_attr = (hs0_ref[...] + hs1_ref[...]) * crec
    ea = jnp.dot(he_attr, linw_ref[...], preferred_element_type=F32)
    b_ref[...] = jnp.dot(ea, atte_ref[...], preferred_element_type=F32)
    binv_ref[...] = jnp.broadcast_to(crec, (BN, 8))
    d = d0_ref[...][:, :1] + d1_ref[...][:, :1]
    drec = jnp.where(d > 0, 1.0 / jnp.where(d > 0, d, 1.0), 0.0)
    dinv_ref[...] = jnp.broadcast_to(drec, (BN, 8))


_k3 = pl.pallas_call(
    _k3_body,
    grid=(GRID,),
    in_specs=[
        pl.BlockSpec((BN, HD), lambda i: (i, 0)),
        pl.BlockSpec((BN, HD), lambda i: (i, 0)),
        pl.BlockSpec((BN, 16), lambda i: (i, 0)),
        pl.BlockSpec((BN, 16), lambda i: (i, 0)),
        pl.BlockSpec((BN, 16), lambda i: (i, 0)),
        pl.BlockSpec((BN, 16), lambda i: (i, 0)),
        _const_spec((HD, XLW)),
        _const_spec((XLW, 16)),
    ],
    out_specs=[
        pl.BlockSpec((BN, 16), lambda i: (i, 0)),
        pl.BlockSpec((BN, 8), lambda i: (i, 0)),
        pl.BlockSpec((BN, 8), lambda i: (i, 0)),
    ],
    out_shape=[
        jax.ShapeDtypeStruct((E, 16), F32),
        jax.ShapeDtypeStruct((E, 8), F32),
        jax.ShapeDtypeStruct((N, 8), F32),
    ],
)


def _k5b_body(p0_ref, p1_ref, binv_ref, oe_ref):
    oe_ref[...] = (p0_ref[...] + p1_ref[...]) * binv_ref[...][:, :1]


_k5b = pl.pallas_call(
    _k5b_body,
    grid=(GRID,),
    in_specs=[
        pl.BlockSpec((BN, XLW), lambda i: (i, 0)),
        pl.BlockSpec((BN, XLW), lambda i: (i, 0)),
        pl.BlockSpec((BN, 8), lambda i: (i, 0)),
    ],
    out_specs=pl.BlockSpec((BN, XLW), lambda i: (i, 0)),
    out_shape=jax.ShapeDtypeStruct((E, XLW), F32),
)


def _k7_body(p0_ref, p1_ref, dinv_ref, z_ref, cb_ref,
             f0w1_ref, f0b1_ref, f0w2t_ref, f0b2_ref,
             f1w1_ref, f1b1_ref, f1w2t_ref, f1b2_ref,
             y0_ref, y1_ref, pcat_ref):
    o = (p0_ref[...] + p1_ref[...]) * dinv_ref[...][:, :1]
    om = (o[:, :GD] + o[:, GD:]) * 0.5 + cb_ref[...]
    ol = jnp.maximum(om, 0.01 * om)
    pcat = jnp.concatenate([z_ref[...], ol], axis=1)
    pcat_ref[...] = pcat
    h0 = jnp.maximum(jnp.dot(pcat, f0w1_ref[...], preferred_element_type=F32)
                     + f0b1_ref[...], 0.0)
    y0_ref[...] = (jnp.sum(h0 * f0w2t_ref[...], axis=1, keepdims=True)
                   + f0b2_ref[...][:, :1])
    h1 = jnp.maximum(jnp.dot(pcat, f1w1_ref[...], preferred_element_type=F32)
                     + f1b1_ref[...], 0.0)
    y1_ref[...] = (jnp.sum(h1 * f1w2t_ref[...], axis=1, keepdims=True)
                   + f1b2_ref[...][:, :1])


YD = HD + GD

_k7 = pl.pallas_call(
    _k7_body,
    grid=(GRID,),
    in_specs=[
        pl.BlockSpec((BN, XLW), lambda i: (i, 0)),
        pl.BlockSpec((BN, XLW), lambda i: (i, 0)),
        pl.BlockSpec((BN, 8), lambda i: (i, 0)),
        pl.BlockSpec((BN, HD), lambda i: (i, 0)),
        _const_spec((1, GD)),
        _const_spec((YD, YD)),
        _const_spec((1, YD)),
        _const_spec((1, YD)),
        _const_spec((1, 1)),
        _const_spec((YD, YD)),
        _const_spec((1, YD)),
        _const_spec((1, YD)),
        _const_spec((1, 1)),
    ],
    out_specs=[
        pl.BlockSpec((BN, 1), lambda i: (i, 0)),
        pl.BlockSpec((BN, 1), lambda i: (i, 0)),
        pl.BlockSpec((BN, YD), lambda i: (i, 0)),
    ],
    out_shape=[
        jax.ShapeDtypeStruct((N, 1), F32),
        jax.ShapeDtypeStruct((N, 1), F32),
        jax.ShapeDtypeStruct((N, YD), F32),
    ],
)


def kernel(X, T, hyperedge_index, cW1, cb1, cW2, cb2, lin_W, att, conv_bias,
           f0W1, f0b1, f0W2, f0b2, f1W1, f1b1, f1W2, f1b2):
    node_idx = hyperedge_index[0]
    he_idx = hyperedge_index[1]
    tf = T.astype(F32).reshape(N, 1)

    attn_pad = jnp.zeros((XLW, 16), F32)
    attn_pad = attn_pad.at[0:GD, 0].set(att[0, :GD]).at[GD:XLW, 1].set(att[1, :GD])
    atte_pad = jnp.zeros((XLW, 16), F32)
    atte_pad = atte_pad.at[0:GD, 0].set(att[0, GD:]).at[GD:XLW, 1].set(att[1, GD:])

    Z, P, XL, a_node, amax8 = _k1(
        X, tf, cW1, cb1.reshape(1, HD), cW2, cb2.reshape(1, HD), lin_W,
        attn_pad)
    amax_vec = amax8[0]

    hes_p, cnt_p, deg_p = _k2(P, node_idx, he_idx)

    b_he, binv, dinv = _k3(hes_p[0], hes_p[1], cnt_p[0], cnt_p[1],
                           deg_p[0], deg_p[1], lin_W, atte_pad)

    ex, esum_p = _k4(a_node, b_he, amax_vec, node_idx, he_idx)

    alpha, oe_p = _k5(XL, ex, esum_p[0], esum_p[1], node_idx, he_idx)

    oe = _k5b(oe_p[0], oe_p[1], binv)

    out_p = _k6(oe, alpha, node_idx, he_idx)

    y0, y1, pcat = _k7(out_p[0], out_p[1], dinv, Z,
                       conv_bias.reshape(1, GD),
                       f0W1, f0b1.reshape(1, YD), f0W2.reshape(1, YD),
                       f0b2.reshape(1, 1),
                       f1W1, f1b1.reshape(1, YD), f1W2.reshape(1, YD),
                       f1b2.reshape(1, 1))
    return y0, y1, pcat


# speedup vs baseline: 38.7505x; 1.1582x over previous
import functools

import jax
import jax.numpy as jnp
from jax import lax
from jax.experimental import pallas as pl
from jax.experimental.pallas import tpu as pltpu
from jax.experimental.pallas import tpu_sc as plsc

N = 10000
E = 10000
I = 320000
NIN = 128
HD = 64
GD = 64
HEADS = 2
XLW = HEADS * GD

NC, NS = 2, 16
NW = NC * NS
PER_TILE = I // NW
C = 80
NCH = PER_TILE // C
RB = 624
ZR = 208
TAIL = N - RB * NS

F32 = jnp.float32
_mesh = plsc.VectorSubcoreMesh(core_axis_name="c", subcore_axis_name="s",
                               num_cores=NC, num_subcores=NS)
_sc_params = pltpu.CompilerParams(use_tc_tiling_on_sc=False)

def _fill(ref, ncol16, val):
    val_vec = jnp.full((16,), val, F32)

    @plsc.parallel_loop(0, ref.shape[0], unroll=4)
    def _(i):
        for t in range(ncol16):
            ref[i, pl.ds(t * 16, 16)] = val_vec


def _worker_id():
    return lax.axis_index("s") * NC + lax.axis_index("c")


def _zero_shared(zb, sh, sid):
    for t in range(3):
        pltpu.sync_copy(zb, sh.at[pl.ds(sid * RB + t * ZR, ZR)])

    @pl.when(sid == 0)
    def _():
        pltpu.sync_copy(zb.at[pl.ds(0, TAIL)], sh.at[pl.ds(RB * NS, TAIL)])


def _writeback(sh, out, cid, sid):
    for t in range(3):
        o = sid * RB + t * ZR
        pltpu.sync_copy(sh.at[pl.ds(o, ZR)], out.at[cid, pl.ds(o, ZR)])

    @pl.when(sid == 0)
    def _():
        pltpu.sync_copy(sh.at[pl.ds(RB * NS, TAIL)],
                        out.at[cid, pl.ds(RB * NS, TAIL)])


def _k2_body(p_hbm, ni_hbm, he_hbm,
             hes_out, cnt_out, deg_out,
             he_sh, cnt_sh, deg_sh,
             zb64, zb16, ones16, ni_v, he_v, rows_v, sem):
    cid = lax.axis_index("c")
    sid = lax.axis_index("s")
    w = _worker_id()
    _fill(zb64, 4, 0.0)
    _fill(zb16, 1, 0.0)
    _fill(ones16, 1, 1.0)
    _zero_shared(zb64, he_sh, sid)
    _zero_shared(zb16, cnt_sh, sid)
    _zero_shared(zb16, deg_sh, sid)
    plsc.subcore_barrier()
    base = w * PER_TILE

    def chunk(g, carry):
        off = base + g * C
        pltpu.sync_copy(ni_hbm.at[pl.ds(off, C)], ni_v)
        pltpu.sync_copy(he_hbm.at[pl.ds(off, C)], he_v)
        pltpu.async_copy(p_hbm.at[ni_v], rows_v, sem).wait()
        pltpu.sync_copy(rows_v, he_sh.at[he_v], add=True)
        pltpu.sync_copy(ones16, cnt_sh.at[he_v], add=True)
        pltpu.sync_copy(ones16, deg_sh.at[ni_v], add=True)
        return carry

    lax.fori_loop(0, NCH, chunk, 0)
    plsc.subcore_barrier()
    _writeback(he_sh, hes_out, cid, sid)
    _writeback(cnt_sh, cnt_out, cid, sid)
    _writeback(deg_sh, deg_out, cid, sid)


_k2 = pl.kernel(
    _k2_body,
    compiler_params=_sc_params,
    out_type=(
        jax.ShapeDtypeStruct((NC, E, HD), F32),
        jax.ShapeDtypeStruct((NC, E, 16), F32),
        jax.ShapeDtypeStruct((NC, N, 16), F32),
    ),
    mesh=_mesh,
    scratch_types=[
        pltpu.VMEM_SHARED((E, HD), F32),
        pltpu.VMEM_SHARED((E, 16), F32),
        pltpu.VMEM_SHARED((N, 16), F32),
        pltpu.VMEM((ZR, HD), F32),
        pltpu.VMEM((ZR, 16), F32),
        pltpu.VMEM((C, 16), F32),
        pltpu.VMEM((C,), jnp.int32),
        pltpu.VMEM((C,), jnp.int32),
        pltpu.VMEM((C, HD), F32),
        pltpu.SemaphoreType.DMA,
    ],
)


def _k4_body(a_hbm, b_hbm, amax_hbm, ni_hbm, he_hbm,
             ex_out, esum_out,
             es_sh, zb16, amax_v, ni_v, he_v, ag, bg, exb, sem):
    cid = lax.axis_index("c")
    sid = lax.axis_index("s")
    w = _worker_id()
    _fill(zb16, 1, 0.0)
    pltpu.sync_copy(amax_hbm, amax_v)
    _zero_shared(zb16, es_sh, sid)
    plsc.subcore_barrier()
    amx = amax_v[...]
    base = w * PER_TILE

    def chunk(g, carry):
        off = base + g * C
        pltpu.sync_copy(ni_hbm.at[pl.ds(off, C)], ni_v)
        pltpu.sync_copy(he_hbm.at[pl.ds(off, C)], he_v)
        d1 = pltpu.async_copy(a_hbm.at[ni_v], ag, sem)
        d2 = pltpu.async_copy(b_hbm.at[he_v], bg, sem)
        d1.wait()
        d2.wait()

        @plsc.parallel_loop(0, C, unroll=4)
        def _(j):
            av = ag[j]
            bv = bg[j]
            s = av + bv
            al = jnp.maximum(s, 0.2 * s)
            m = bv + amx
            ml = jnp.maximum(m, 0.2 * m)
            exb[j] = jnp.exp(al - ml)
        pltpu.sync_copy(exb, ex_out.at[pl.ds(off, C)])
        pltpu.sync_copy(exb, es_sh.at[he_v], add=True)
        return carry

    lax.fori_loop(0, NCH, chunk, 0)
    plsc.subcore_barrier()
    _writeback(es_sh, esum_out, cid, sid)


_k4 = pl.kernel(
    _k4_body,
    compiler_params=_sc_params,
    out_type=(
        jax.ShapeDtypeStruct((I, 16), F32),
        jax.ShapeDtypeStruct((NC, E, 16), F32),
    ),
    mesh=_mesh,
    scratch_types=[
        pltpu.VMEM_SHARED((E, 16), F32),
        pltpu.VMEM((ZR, 16), F32),
        pltpu.VMEM((16,), F32),
        pltpu.VMEM((C,), jnp.int32),
        pltpu.VMEM((C,), jnp.int32),
        pltpu.VMEM((C, 16), F32),
        pltpu.VMEM((C, 16), F32),
        pltpu.VMEM((C, 16), F32),
        pltpu.SemaphoreType.DMA,
    ],
)


def _k5_body(xl_hbm, ex_hbm, es0_hbm, es1_hbm, ni_hbm, he_hbm,
             alpha_out, oe_out,
             oe_sh, zb, ni_v, he_v, xg, s0, s1, exv, alb, sem):
    cid = lax.axis_index("c")
    sid = lax.axis_index("s")
    w = _worker_id()
    _fill(zb, 8, 0.0)
    _zero_shared(zb, oe_sh, sid)
    plsc.subcore_barrier()
    base = w * PER_TILE

    def chunk(g, carry):
        off = base + g * C
        pltpu.sync_copy(ni_hbm.at[pl.ds(off, C)], ni_v)
        pltpu.sync_copy(he_hbm.at[pl.ds(off, C)], he_v)
        d1 = pltpu.async_copy(xl_hbm.at[ni_v], xg, sem)
        d2 = pltpu.async_copy(es0_hbm.at[he_v], s0, sem)
        d3 = pltpu.async_copy(es1_hbm.at[he_v], s1, sem)
        pltpu.sync_copy(ex_hbm.at[pl.ds(off, C)], exv)
        d1.wait()
        d2.wait()
        d3.wait()

        @plsc.parallel_loop(0, C, unroll=4)
        def _(j):
            al = exv[j] / (s0[j] + s1[j] + 1e-16)
            alb[j] = al
            b0 = jnp.full((16,), al[0], F32)
            b1 = jnp.full((16,), al[1], F32)
            for t in range(8):
                bv = b0 if t < 4 else b1
                xg[j, pl.ds(t * 16, 16)] = xg[j, pl.ds(t * 16, 16)] * bv
        pltpu.sync_copy(alb, alpha_out.at[pl.ds(off, C)])
        pltpu.sync_copy(xg, oe_sh.at[he_v], add=True)
        return carry

    lax.fori_loop(0, NCH, chunk, 0)
    plsc.subcore_barrier()
    _writeback(oe_sh, oe_out, cid, sid)


_k5 = pl.kernel(
    _k5_body,
    compiler_params=_sc_params,
    out_type=(
        jax.ShapeDtypeStruct((I, 16), F32),
        jax.ShapeDtypeStruct((NC, E, XLW), F32),
    ),
    mesh=_mesh,
    scratch_types=[
        pltpu.VMEM_SHARED((E, XLW), F32),
        pltpu.VMEM((ZR, XLW), F32),
        pltpu.VMEM((C,), jnp.int32),
        pltpu.VMEM((C,), jnp.int32),
        pltpu.VMEM((C, XLW), F32),
        pltpu.VMEM((C, 16), F32),
        pltpu.VMEM((C, 16), F32),
        pltpu.VMEM((C, 16), F32),
        pltpu.VMEM((C, 16), F32),
        pltpu.SemaphoreType.DMA,
    ],
)


def _k6_body(oe_hbm, al_hbm, ni_hbm, he_hbm,
             out_out,
             o_sh, zb, ni_v, he_v, xg, alb, sem):
    cid = lax.axis_index("c")
    sid = lax.axis_index("s")
    w = _worker_id()
    _fill(zb, 8, 0.0)
    _zero_shared(zb, o_sh, sid)
    plsc.subcore_barrier()
    base = w * PER_TILE

    def chunk(g, carry):
        off = base + g * C
        pltpu.sync_copy(ni_hbm.at[pl.ds(off, C)], ni_v)
        pltpu.sync_copy(he_hbm.at[pl.ds(off, C)], he_v)
        d1 = pltpu.async_copy(oe_hbm.at[he_v], xg, sem)
        pltpu.sync_copy(al_hbm.at[pl.ds(off, C)], alb)
        d1.wait()

        @plsc.parallel_loop(0, C, unroll=4)
        def _(j):
            al = alb[j]
            b0 = jnp.full((16,), al[0], F32)
            b1 = jnp.full((16,), al[1], F32)
            for t in range(8):
                bv = b0 if t < 4 else b1
                xg[j, pl.ds(t * 16, 16)] = xg[j, pl.ds(t * 16, 16)] * bv
        pltpu.sync_copy(xg, o_sh.at[ni_v], add=True)
        return carry

    lax.fori_loop(0, NCH, chunk, 0)
    plsc.subcore_barrier()
    _writeback(o_sh, out_out, cid, sid)


_k6 = pl.kernel(
    _k6_body,
    compiler_params=_sc_params,
    out_type=jax.ShapeDtypeStruct((NC, N, XLW), F32),
    mesh=_mesh,
    scratch_types=[
        pltpu.VMEM_SHARED((N, XLW), F32),
        pltpu.VMEM((ZR, XLW), F32),
        pltpu.VMEM((C,), jnp.int32),
        pltpu.VMEM((C,), jnp.int32),
        pltpu.VMEM((C, XLW), F32),
        pltpu.VMEM((C, 16), F32),
        pltpu.SemaphoreType.DMA,
    ],
)


BN = 2000
GRID = N // BN


def _k1_body(x_ref, tf_ref, cw1_ref, cb1_ref, cw2_ref, cb2_ref, linw_ref,
             attn_ref, z_ref, p_ref, xl_ref, an_ref, amax_ref):
    x = x_ref[...]
    z = jnp.maximum(jnp.dot(x, cw1_ref[...], preferred_element_type=F32)
                    + cb1_ref[...], 0.0)
    z = jnp.dot(z, cw2_ref[...], preferred_element_type=F32) + cb2_ref[...]
    z_ref[...] = z
    p = z * tf_ref[...]
    p_ref[...] = p
    xl = jnp.dot(p, linw_ref[...], preferred_element_type=F32)
    xl_ref[...] = xl
    an = jnp.dot(xl, attn_ref[...], preferred_element_type=F32)
    an_ref[...] = an
    cur = jnp.broadcast_to(jnp.max(an, axis=0, keepdims=True), (8, 16))

    @pl.when(pl.program_id(0) == 0)
    def _init():
        amax_ref[...] = cur

    @pl.when(pl.program_id(0) > 0)
    def _acc():
        amax_ref[...] = jnp.maximum(amax_ref[...], cur)


def _const_spec(shape):
    return pl.BlockSpec(shape, lambda i: (0, 0))


_k1 = pl.pallas_call(
    _k1_body,
    grid=(GRID,),
    in_specs=[
        pl.BlockSpec((BN, NIN), lambda i: (i, 0)),
        pl.BlockSpec((BN, 1), lambda i: (i, 0)),
        _const_spec((NIN, HD)),
        _const_spec((1, HD)),
        _const_spec((HD, HD)),
        _const_spec((1, HD)),
        _const_spec((HD, XLW)),
        _const_spec((XLW, 16)),
    ],
    out_specs=[
        pl.BlockSpec((BN, HD), lambda i: (i, 0)),
        pl.BlockSpec((BN, HD), lambda i: (i, 0)),
        pl.BlockSpec((BN, XLW), lambda i: (i, 0)),
        pl.BlockSpec((BN, 16), lambda i: (i, 0)),
        _const_spec((8, 16)),
    ],
    out_shape=[
        jax.ShapeDtypeStruct((N, HD), F32),
        jax.ShapeDtypeStruct((N, HD), F32),
        jax.ShapeDtypeStruct((N, XLW), F32),
        jax.ShapeDtypeStruct((N, 16), F32),
        jax.ShapeDtypeStruct((8, 16), F32),
    ],
)


def _k3_body(hs0_ref, hs1_ref, c0_ref, c1_ref, d0_ref, d1_ref, linw_ref,
             atte_ref, b_ref, binv_ref, dinv_ref):
    c = c0_ref[...][:, :1] + c1_ref[...][:, :1]
    crec = jnp.where(c > 0, 1.0 / jnp.where(c > 0, c, 1.0), 0.0)
    he_attr = (hs0_ref[...] + hs1_ref[...]) * crec
    ea = jnp.dot(he_attr, linw_ref[...], preferred_element_type=F32)
    b_ref[...] = jnp.dot(ea, atte_ref[...], preferred_element_type=F32)
    binv_ref[...] = jnp.broadcast_to(crec, (BN, 8))
    d = d0_ref[...][:, :1] + d1_ref[...][:, :1]
    drec = jnp.where(d > 0, 1.0 / jnp.where(d > 0, d, 1.0), 0.0)
    dinv_ref[...] = jnp.broadcast_to(drec, (BN, 8))


_k3 = pl.pallas_call(
    _k3_body,
    grid=(GRID,),
    in_specs=[
        pl.BlockSpec((BN, HD), lambda i: (i, 0)),
        pl.BlockSpec((BN, HD), lambda i: (i, 0)),
        pl.BlockSpec((BN, 16), lambda i: (i, 0)),
        pl.BlockSpec((BN, 16), lambda i: (i, 0)),
        pl.BlockSpec((BN, 16), lambda i: (i, 0)),
        pl.BlockSpec((BN, 16), lambda i: (i, 0)),
        _const_spec((HD, XLW)),
        _const_spec((XLW, 16)),
    ],
    out_specs=[
        pl.BlockSpec((BN, 16), lambda i: (i, 0)),
        pl.BlockSpec((BN, 8), lambda i: (i, 0)),
        pl.BlockSpec((BN, 8), lambda i: (i, 0)),
    ],
    out_shape=[
        jax.ShapeDtypeStruct((E, 16), F32),
        jax.ShapeDtypeStruct((E, 8), F32),
        jax.ShapeDtypeStruct((N, 8), F32),
    ],
)


def _k5b_body(p0_ref, p1_ref, binv_ref, oe_ref):
    oe_ref[...] = (p0_ref[...] + p1_ref[...]) * binv_ref[...][:, :1]


_k5b = pl.pallas_call(
    _k5b_body,
    grid=(GRID,),
    in_specs=[
        pl.BlockSpec((BN, XLW), lambda i: (i, 0)),
        pl.BlockSpec((BN, XLW), lambda i: (i, 0)),
        pl.BlockSpec((BN, 8), lambda i: (i, 0)),
    ],
    out_specs=pl.BlockSpec((BN, XLW), lambda i: (i, 0)),
    out_shape=jax.ShapeDtypeStruct((E, XLW), F32),
)


def _k7_body(p0_ref, p1_ref, dinv_ref, z_ref, cb_ref,
             f0w1_ref, f0b1_ref, f0w2t_ref, f0b2_ref,
             f1w1_ref, f1b1_ref, f1w2t_ref, f1b2_ref,
             y0_ref, y1_ref, pcat_ref):
    o = (p0_ref[...] + p1_ref[...]) * dinv_ref[...][:, :1]
    om = (o[:, :GD] + o[:, GD:]) * 0.5 + cb_ref[...]
    ol = jnp.maximum(om, 0.01 * om)
    pcat = jnp.concatenate([z_ref[...], ol], axis=1)
    pcat_ref[...] = pcat
    h0 = jnp.maximum(jnp.dot(pcat, f0w1_ref[...], preferred_element_type=F32)
                     + f0b1_ref[...], 0.0)
    y0_ref[...] = (jnp.sum(h0 * f0w2t_ref[...], axis=1, keepdims=True)
                   + f0b2_ref[...][:, :1])
    h1 = jnp.maximum(jnp.dot(pcat, f1w1_ref[...], preferred_element_type=F32)
                     + f1b1_ref[...], 0.0)
    y1_ref[...] = (jnp.sum(h1 * f1w2t_ref[...], axis=1, keepdims=True)
                   + f1b2_ref[...][:, :1])


YD = HD + GD

_k7 = pl.pallas_call(
    _k7_body,
    grid=(GRID,),
    in_specs=[
        pl.BlockSpec((BN, XLW), lambda i: (i, 0)),
        pl.BlockSpec((BN, XLW), lambda i: (i, 0)),
        pl.BlockSpec((BN, 8), lambda i: (i, 0)),
        pl.BlockSpec((BN, HD), lambda i: (i, 0)),
        _const_spec((1, GD)),
        _const_spec((YD, YD)),
        _const_spec((1, YD)),
        _const_spec((1, YD)),
        _const_spec((1, 1)),
        _const_spec((YD, YD)),
        _const_spec((1, YD)),
        _const_spec((1, YD)),
        _const_spec((1, 1)),
    ],
    out_specs=[
        pl.BlockSpec((BN, 1), lambda i: (i, 0)),
        pl.BlockSpec((BN, 1), lambda i: (i, 0)),
        pl.BlockSpec((BN, YD), lambda i: (i, 0)),
    ],
    out_shape=[
        jax.ShapeDtypeStruct((N, 1), F32),
        jax.ShapeDtypeStruct((N, 1), F32),
        jax.ShapeDtypeStruct((N, YD), F32),
    ],
)


def kernel(X, T, hyperedge_index, cW1, cb1, cW2, cb2, lin_W, att, conv_bias,
           f0W1, f0b1, f0W2, f0b2, f1W1, f1b1, f1W2, f1b2):
    node_idx = hyperedge_index[0]
    he_idx = hyperedge_index[1]
    tf = T.astype(F32).reshape(N, 1)

    attn_pad = jnp.zeros((XLW, 16), F32)
    attn_pad = attn_pad.at[0:GD, 0].set(att[0, :GD]).at[GD:XLW, 1].set(att[1, :GD])
    atte_pad = jnp.zeros((XLW, 16), F32)
    atte_pad = atte_pad.at[0:GD, 0].set(att[0, GD:]).at[GD:XLW, 1].set(att[1, GD:])

    Z, P, XL, a_node, amax8 = _k1(
        X, tf, cW1, cb1.reshape(1, HD), cW2, cb2.reshape(1, HD), lin_W,
        attn_pad)
    amax_vec = amax8[0]

    hes_p, cnt_p, deg_p = _k2(P, node_idx, he_idx)

    b_he, binv, dinv = _k3(hes_p[0], hes_p[1], cnt_p[0], cnt_p[1],
                           deg_p[0], deg_p[1], lin_W, atte_pad)

    ex, esum_p = _k4(a_node, b_he, amax_vec, node_idx, he_idx)

    alpha, oe_p = _k5(XL, ex, esum_p[0], esum_p[1], node_idx, he_idx)

    oe = _k5b(oe_p[0], oe_p[1], binv)

    out_p = _k6(oe, alpha, node_idx, he_idx)

    y0, y1, pcat = _k7(out_p[0], out_p[1], dinv, Z,
                       conv_bias.reshape(1, GD),
                       f0W1, f0b1.reshape(1, YD), f0W2.reshape(1, YD),
                       f0b2.reshape(1, 1),
                       f1W1, f1b1.reshape(1, YD), f1W2.reshape(1, YD),
                       f1b2.reshape(1, 1))
    return y0, y1, pcat
